# Initial kernel scaffold; baseline (speedup 1.0000x reference)
import functools

import jax
import jax.numpy as jnp
from jax import lax
from jax.experimental import pallas as pl
from jax.experimental.pallas import tpu as pltpu
from jax.experimental.pallas import tpu_sc as plsc

N = 10000
E = 320000
D = 128
H = 8
C = 128
HP = 16
NEG = 0.2

NC = 2
NS = 16
NW = NC * NS
EPW = E // NW
CH = 80
NCHUNK = EPW // CH
RPT = N // NS

BN = 2000


def _proj_body(x_ref, w_ref, as_ref, ad_ref,
               xp_ref, ats_ref, atd_ref, ks_ref, kd_ref):
    i = pl.program_id(0)
    xp = jnp.dot(x_ref[...], w_ref[...], preferred_element_type=jnp.float32)
    xp_ref[...] = xp
    xph = xp.reshape(BN, H, C)
    asb = jnp.sum(xph * as_ref[...][None], axis=-1)
    adb = jnp.sum(xph * ad_ref[...][None], axis=-1)
    pad = jnp.zeros((BN, HP - H), jnp.float32)
    asbp = jnp.concatenate([asb, pad], axis=1)
    adbp = jnp.concatenate([adb, pad], axis=1)
    ats_ref[...] = asbp
    atd_ref[...] = adbp

    @pl.when(i == 0)
    def _():
        ks_ref[...] = jnp.full((1, HP), -1e30, jnp.float32)
        kd_ref[...] = jnp.full((1, HP), -1e30, jnp.float32)

    ks_ref[...] = jnp.maximum(ks_ref[...], jnp.max(asbp, axis=0, keepdims=True))
    kd_ref[...] = jnp.maximum(kd_ref[...], jnp.max(adbp, axis=0, keepdims=True))


def _proj(x, w, att_src, att_dst):
    return pl.pallas_call(
        _proj_body,
        grid=(N // BN,),
        in_specs=[
            pl.BlockSpec((BN, D), lambda i: (i, 0)),
            pl.BlockSpec((D, H * C), lambda i: (0, 0)),
            pl.BlockSpec((H, C), lambda i: (0, 0)),
            pl.BlockSpec((H, C), lambda i: (0, 0)),
        ],
        out_specs=[
            pl.BlockSpec((BN, H * C), lambda i: (i, 0)),
            pl.BlockSpec((BN, HP), lambda i: (i, 0)),
            pl.BlockSpec((BN, HP), lambda i: (i, 0)),
            pl.BlockSpec((1, HP), lambda i: (0, 0)),
            pl.BlockSpec((1, HP), lambda i: (0, 0)),
        ],
        out_shape=[
            jax.ShapeDtypeStruct((N, H * C), jnp.float32),
            jax.ShapeDtypeStruct((N, HP), jnp.float32),
            jax.ShapeDtypeStruct((N, HP), jnp.float32),
            jax.ShapeDtypeStruct((1, HP), jnp.float32),
            jax.ShapeDtypeStruct((1, HP), jnp.float32),
        ],
    )(x, w, att_src, att_dst)


_MESH = plsc.VectorSubcoreMesh(core_axis_name="c", subcore_axis_name="s")


@functools.partial(
    pl.kernel,
    out_type=[
        jax.ShapeDtypeStruct((E, HP), jnp.float32),
        jax.ShapeDtypeStruct((NC, N, HP), jnp.float32),
    ],
    mesh=_MESH,
    scratch_types=[
        pltpu.VMEM_SHARED((N, HP), jnp.float32),
        pltpu.VMEM((CH,), jnp.int32),
        pltpu.VMEM((CH,), jnp.int32),
        pltpu.VMEM((CH, HP), jnp.float32),
        pltpu.VMEM((CH, HP), jnp.float32),
        pltpu.VMEM((CH, HP), jnp.float32),
        pltpu.VMEM((1, HP), jnp.float32),
        pltpu.VMEM((1, HP), jnp.float32),
    ],
)
def _edge_pass_a(src_hbm, dst_hbm, ats_hbm, atd_hbm, ks_hbm, kd_hbm, z16_hbm,
                 e_hbm, den_hbm,
                 den_sh, sidx, didx, ag, bg, ebuf, ksb, kdb):
    c = lax.axis_index("c")
    s = lax.axis_index("s")
    wid = c * NS + s
    base = wid * EPW

    pltpu.sync_copy(z16_hbm.at[pl.ds(s * RPT, RPT), :],
                    den_sh.at[pl.ds(s * RPT, RPT), :])
    plsc.subcore_barrier()

    pltpu.sync_copy(ks_hbm, ksb)
    pltpu.sync_copy(kd_hbm, kdb)
    ksum = ksb[0] + kdb[0]
    kvec = jnp.maximum(ksum, NEG * ksum)

    def chunk(ci, carry):
        off = base + ci * CH
        pltpu.sync_copy(src_hbm.at[pl.ds(off, CH)], sidx)
        pltpu.sync_copy(dst_hbm.at[pl.ds(off, CH)], didx)
        pltpu.sync_copy(ats_hbm.at[sidx], ag)
        pltpu.sync_copy(atd_hbm.at[didx], bg)

        def edge(i, _):
            a = ag[i] + bg[i]
            a = jnp.maximum(a, NEG * a)
            ebuf[i] = jnp.exp(a - kvec)
            return 0

        lax.fori_loop(0, CH, edge, 0)
        pltpu.sync_copy(ebuf, e_hbm.at[pl.ds(off, CH), :])
        pltpu.sync_copy(ebuf, den_sh.at[didx], add=True)
        return carry

    lax.fori_loop(0, NCHUNK, chunk, 0)

    plsc.subcore_barrier()
    pltpu.sync_copy(den_sh.at[pl.ds(s * RPT, RPT), :],
                    den_hbm.at[c].at[pl.ds(s * RPT, RPT), :])


@functools.partial(
    pl.kernel,
    out_type=jax.ShapeDtypeStruct((NC, N, C), jnp.float32),
    mesh=_MESH,
    scratch_types=[
        pltpu.VMEM_SHARED((N, C), jnp.float32),
        pltpu.VMEM((CH,), jnp.int32),
        pltpu.VMEM((CH,), jnp.int32),
        pltpu.VMEM((CH, HP), jnp.float32),
        pltpu.VMEM((CH, HP), jnp.float32),
        pltpu.VMEM((CH, HP), jnp.float32),
        pltpu.VMEM((CH, H, C), jnp.float32),
        pltpu.VMEM((CH, C), jnp.float32),
    ],
)
def _edge_pass_b(src_hbm, dst_hbm, e_hbm, d0_hbm, d1_hbm, xp_hbm, z128_hbm,
                 out_hbm,
                 out_sh, sidx, didx, ebuf, d0b, d1b, xpb, msgb):
    c = lax.axis_index("c")
    s = lax.axis_index("s")
    wid = c * NS + s
    base = wid * EPW

    pltpu.sync_copy(z128_hbm.at[pl.ds(s * RPT, RPT), :],
                    out_sh.at[pl.ds(s * RPT, RPT), :])
    plsc.subcore_barrier()

    def chunk(ci, carry):
        off = base + ci * CH
        pltpu.sync_copy(src_hbm.at[pl.ds(off, CH)], sidx)
        pltpu.sync_copy(dst_hbm.at[pl.ds(off, CH)], didx)
        pltpu.sync_copy(e_hbm.at[pl.ds(off, CH), :], ebuf)
        pltpu.sync_copy(d0_hbm.at[didx], d0b)
        pltpu.sync_copy(d1_hbm.at[didx], d1b)
        pltpu.sync_copy(xp_hbm.at[sidx], xpb)

        def edge(i, _):
            accs = [jnp.zeros((16,), jnp.float32) for _ in range(C // 16)]
            for h in range(H):
                ev = ebuf[i, h]
                dv = d0b[i, h] + d1b[i, h] + 1e-16
                cf = ev / dv
                bs = jnp.full((16,), cf, jnp.float32)
                for cc in range(C // 16):
                    accs[cc] = accs[cc] + bs * xpb[i, h, pl.ds(cc * 16, 16)]
            for cc in range(C // 16):
                msgb[i, pl.ds(cc * 16, 16)] = accs[cc]
            return 0

        lax.fori_loop(0, CH, edge, 0)
        pltpu.sync_copy(msgb, out_sh.at[didx], add=True)
        return carry

    lax.fori_loop(0, NCHUNK, chunk, 0)

    plsc.subcore_barrier()
    pltpu.sync_copy(out_sh.at[pl.ds(s * RPT, RPT), :],
                    out_hbm.at[c].at[pl.ds(s * RPT, RPT), :])


def _final_body(p_ref, b_ref, o_ref):
    o_ref[...] = (p_ref[0] + p_ref[1]) * (1.0 / H) + b_ref[...]


def _final(partials, bias2d):
    return pl.pallas_call(
        _final_body,
        grid=(N // BN,),
        in_specs=[
            pl.BlockSpec((NC, BN, C), lambda i: (0, i, 0)),
            pl.BlockSpec((1, C), lambda i: (0, 0)),
        ],
        out_specs=pl.BlockSpec((BN, C), lambda i: (i, 0)),
        out_shape=jax.ShapeDtypeStruct((N, C), jnp.float32),
    )(partials, bias2d)


def kernel(x, edge_index, W, att_src, att_dst, bias):
    src = edge_index[0]
    dst = edge_index[1]
    xp, ats, atd, ks, kd = _proj(x, W, att_src, att_dst)
    z16 = jnp.zeros((N, HP), jnp.float32)
    z128 = jnp.zeros((N, C), jnp.float32)
    e, den = _edge_pass_a(src, dst, ats, atd, ks, kd, z16)
    out_p = _edge_pass_b(src, dst, e, den[0], den[1],
                         xp.reshape(N, H, C), z128)
    return _final(out_p, bias.reshape(1, C))


# speedup vs baseline: 16.6151x; 16.6151x over previous
import functools

import jax
import jax.numpy as jnp
from jax import lax
from jax.experimental import pallas as pl
from jax.experimental.pallas import tpu as pltpu
from jax.experimental.pallas import tpu_sc as plsc

N = 10000
E = 320000
D = 128
H = 8
C = 128
HP = 16
NEG = 0.2

NC = 2
NS = 16
NW = NC * NS
EPW = E // NW
CH = 40
NCHUNK = EPW // CH
NPAD = 10240
RPT = NPAD // NS

BN = 2000


def _proj_body(x_ref, w_ref, as_ref, ad_ref,
               xp_ref, ats_ref, atd_ref, ks_ref, kd_ref):
    i = pl.program_id(0)
    xp = jnp.dot(x_ref[...], w_ref[...], preferred_element_type=jnp.float32)
    xp_ref[...] = xp
    xph = xp.reshape(BN, H, C)
    asb = jnp.sum(xph * as_ref[...][None], axis=-1)
    adb = jnp.sum(xph * ad_ref[...][None], axis=-1)
    pad = jnp.zeros((BN, HP - H), jnp.float32)
    asbp = jnp.concatenate([asb, pad], axis=1)
    adbp = jnp.concatenate([adb, pad], axis=1)
    ats_ref[...] = asbp
    atd_ref[...] = adbp

    @pl.when(i == 0)
    def _():
        ks_ref[...] = jnp.full((1, HP), -1e30, jnp.float32)
        kd_ref[...] = jnp.full((1, HP), -1e30, jnp.float32)

    ks_ref[...] = jnp.maximum(ks_ref[...], jnp.max(asbp, axis=0, keepdims=True))
    kd_ref[...] = jnp.maximum(kd_ref[...], jnp.max(adbp, axis=0, keepdims=True))


def _proj(x, w, att_src, att_dst):
    return pl.pallas_call(
        _proj_body,
        grid=(N // BN,),
        in_specs=[
            pl.BlockSpec((BN, D), lambda i: (i, 0)),
            pl.BlockSpec((D, H * C), lambda i: (0, 0)),
            pl.BlockSpec((H, C), lambda i: (0, 0)),
            pl.BlockSpec((H, C), lambda i: (0, 0)),
        ],
        out_specs=[
            pl.BlockSpec((BN, H * C), lambda i: (i, 0)),
            pl.BlockSpec((BN, HP), lambda i: (i, 0)),
            pl.BlockSpec((BN, HP), lambda i: (i, 0)),
            pl.BlockSpec((1, HP), lambda i: (0, 0)),
            pl.BlockSpec((1, HP), lambda i: (0, 0)),
        ],
        out_shape=[
            jax.ShapeDtypeStruct((N, H * C), jnp.float32),
            jax.ShapeDtypeStruct((N, HP), jnp.float32),
            jax.ShapeDtypeStruct((N, HP), jnp.float32),
            jax.ShapeDtypeStruct((1, HP), jnp.float32),
            jax.ShapeDtypeStruct((1, HP), jnp.float32),
        ],
    )(x, w, att_src, att_dst)


_MESH = plsc.VectorSubcoreMesh(core_axis_name="c", subcore_axis_name="s")


@functools.partial(
    pl.kernel,
    out_type=[
        jax.ShapeDtypeStruct((E, HP), jnp.float32),
        jax.ShapeDtypeStruct((NC, NPAD, HP), jnp.float32),
    ],
    mesh=_MESH,
    compiler_params=pltpu.CompilerParams(use_tc_tiling_on_sc=False),
    scratch_types=[
        pltpu.VMEM_SHARED((NPAD, HP), jnp.float32),
        pltpu.VMEM((CH,), jnp.int32),
        pltpu.VMEM((CH,), jnp.int32),
        pltpu.VMEM((CH, HP), jnp.float32),
        pltpu.VMEM((CH, HP), jnp.float32),
        pltpu.VMEM((CH, HP), jnp.float32),
        pltpu.VMEM((1, HP), jnp.float32),
        pltpu.VMEM((1, HP), jnp.float32),
    ],
)
def _edge_pass_a(src_hbm, dst_hbm, ats_hbm, atd_hbm, ks_hbm, kd_hbm, z16_hbm,
                 e_hbm, den_hbm,
                 den_sh, sidx, didx, ag, bg, ebuf, ksb, kdb):
    c = lax.axis_index("c")
    s = lax.axis_index("s")
    wid = c * NS + s
    base = wid * EPW

    pltpu.sync_copy(z16_hbm.at[pl.ds(s * RPT, RPT), :],
                    den_sh.at[pl.ds(s * RPT, RPT), :])
    plsc.subcore_barrier()

    pltpu.sync_copy(ks_hbm, ksb)
    pltpu.sync_copy(kd_hbm, kdb)
    ksum = ksb[0] + kdb[0]
    kvec = jnp.maximum(ksum, NEG * ksum)

    def chunk(ci, carry):
        off = base + ci * CH
        pltpu.sync_copy(src_hbm.at[pl.ds(off, CH)], sidx)
        pltpu.sync_copy(dst_hbm.at[pl.ds(off, CH)], didx)
        pltpu.sync_copy(ats_hbm.at[sidx], ag)
        pltpu.sync_copy(atd_hbm.at[didx], bg)

        def edge(i, _):
            a = ag[i] + bg[i]
            a = jnp.maximum(a, NEG * a)
            ebuf[i] = jnp.exp(a - kvec)
            return 0

        lax.fori_loop(0, CH, edge, 0)
        pltpu.sync_copy(ebuf, e_hbm.at[pl.ds(off, CH), :])
        pltpu.sync_copy(ebuf, den_sh.at[didx], add=True)
        return carry

    lax.fori_loop(0, NCHUNK, chunk, 0)

    plsc.subcore_barrier()
    pltpu.sync_copy(den_sh.at[pl.ds(s * RPT, RPT), :],
                    den_hbm.at[c].at[pl.ds(s * RPT, RPT), :])


@functools.partial(
    pl.kernel,
    out_type=jax.ShapeDtypeStruct((NC, NPAD, C), jnp.float32),
    mesh=_MESH,
    compiler_params=pltpu.CompilerParams(use_tc_tiling_on_sc=False),
    scratch_types=[
        pltpu.VMEM_SHARED((NPAD, C), jnp.float32),
        pltpu.VMEM((CH,), jnp.int32),
        pltpu.VMEM((CH,), jnp.int32),
        pltpu.VMEM((CH, HP), jnp.float32),
        pltpu.VMEM((CH, HP), jnp.float32),
        pltpu.VMEM((CH, HP), jnp.float32),
        pltpu.VMEM((CH, H, C), jnp.float32),
        pltpu.VMEM((CH, C), jnp.float32),
    ],
)
def _edge_pass_b(src_hbm, dst_hbm, e_hbm, d0_hbm, d1_hbm, xp_hbm, z128_hbm,
                 out_hbm,
                 out_sh, sidx, didx, ebuf, d0b, d1b, xpb, msgb):
    c = lax.axis_index("c")
    s = lax.axis_index("s")
    wid = c * NS + s
    base = wid * EPW

    pltpu.sync_copy(z128_hbm.at[pl.ds(s * RPT, RPT), :],
                    out_sh.at[pl.ds(s * RPT, RPT), :])
    plsc.subcore_barrier()

    def chunk(ci, carry):
        off = base + ci * CH
        pltpu.sync_copy(src_hbm.at[pl.ds(off, CH)], sidx)
        pltpu.sync_copy(dst_hbm.at[pl.ds(off, CH)], didx)
        pltpu.sync_copy(e_hbm.at[pl.ds(off, CH), :], ebuf)
        pltpu.sync_copy(d0_hbm.at[didx], d0b)
        pltpu.sync_copy(d1_hbm.at[didx], d1b)
        pltpu.sync_copy(xp_hbm.at[sidx], xpb)

        def edge(i, _):
            cfv = ebuf[i] / (d0b[i] + d1b[i] + 1e-16)
            accs = [jnp.zeros((16,), jnp.float32) for _ in range(C // 16)]
            for h in range(H):
                bs = jnp.full((16,), cfv[h], jnp.float32)
                for cc in range(C // 16):
                    accs[cc] = accs[cc] + bs * xpb[i, h, pl.ds(cc * 16, 16)]
            for cc in range(C // 16):
                msgb[i, pl.ds(cc * 16, 16)] = accs[cc]
            return 0

        lax.fori_loop(0, CH, edge, 0)
        pltpu.sync_copy(msgb, out_sh.at[didx], add=True)
        return carry

    lax.fori_loop(0, NCHUNK, chunk, 0)

    plsc.subcore_barrier()
    pltpu.sync_copy(out_sh.at[pl.ds(s * RPT, RPT), :],
                    out_hbm.at[c].at[pl.ds(s * RPT, RPT), :])


def _final_body(p_ref, b_ref, o_ref):
    o_ref[...] = (p_ref[0] + p_ref[1]) * (1.0 / H) + b_ref[...]


def _final(partials, bias2d):
    return pl.pallas_call(
        _final_body,
        grid=(N // BN,),
        in_specs=[
            pl.BlockSpec((NC, BN, C), lambda i: (0, i, 0)),
            pl.BlockSpec((1, C), lambda i: (0, 0)),
        ],
        out_specs=pl.BlockSpec((BN, C), lambda i: (i, 0)),
        out_shape=jax.ShapeDtypeStruct((N, C), jnp.float32),
    )(partials, bias2d)


def kernel(x, edge_index, W, att_src, att_dst, bias):
    src = edge_index[0]
    dst = edge_index[1]
    xp, ats, atd, ks, kd = _proj(x, W, att_src, att_dst)
    z16 = jnp.zeros((NPAD, HP), jnp.float32)
    z128 = jnp.zeros((NPAD, C), jnp.float32)
    e, den = _edge_pass_a(src, dst, ats, atd, ks, kd, z16)
    out_p = _edge_pass_b(src, dst, e, den[0], den[1],
                         xp.reshape(N, H, C), z128)
    return _final(out_p, bias.reshape(1, C))


# speedup vs baseline: 33.3337x; 2.0062x over previous
import functools

import jax
import jax.numpy as jnp
import numpy as np
from jax import lax
from jax.experimental import pallas as pl
from jax.experimental.pallas import tpu as pltpu
from jax.experimental.pallas import tpu_sc as plsc

N = 10000
E = 320000
D = 128
H = 8
C = 128
HP = 16
NEG = 0.2

NC = 2
NS = 16
NW = NC * NS
EPW = E // NW
CHA = 80
NCHA = EPW // CHA
CHB = 40
NCHB = EPW // CHB
NPAD = 10240
RPT = NPAD // NS

BN = 2000

_PERM = np.arange(C).reshape(4, 2, 16).transpose(0, 2, 1).reshape(C)


def _proj_body(x_ref, w_ref, as_ref, ad_ref,
               xpbf_ref, ats_ref, atd_ref, ks_ref, kd_ref):
    i = pl.program_id(0)
    xp = jnp.dot(x_ref[...], w_ref[...], preferred_element_type=jnp.float32)
    xpbf_ref[...] = xp.astype(jnp.bfloat16)
    xph = xp.reshape(BN, H, C)
    asb = jnp.sum(xph * as_ref[...][None], axis=-1)
    adb = jnp.sum(xph * ad_ref[...][None], axis=-1)
    pad = jnp.zeros((BN, HP - H), jnp.float32)
    asbp = jnp.concatenate([asb, pad], axis=1)
    adbp = jnp.concatenate([adb, pad], axis=1)
    ats_ref[...] = asbp
    atd_ref[...] = adbp

    @pl.when(i == 0)
    def _():
        ks_ref[...] = jnp.full((1, HP), -1e30, jnp.float32)
        kd_ref[...] = jnp.full((1, HP), -1e30, jnp.float32)

    ks_ref[...] = jnp.maximum(ks_ref[...], jnp.max(asbp, axis=0, keepdims=True))
    kd_ref[...] = jnp.maximum(kd_ref[...], jnp.max(adbp, axis=0, keepdims=True))


def _proj(x, w, att_src, att_dst):
    return pl.pallas_call(
        _proj_body,
        grid=(N // BN,),
        in_specs=[
            pl.BlockSpec((BN, D), lambda i: (i, 0)),
            pl.BlockSpec((D, H * C), lambda i: (0, 0)),
            pl.BlockSpec((H, C), lambda i: (0, 0)),
            pl.BlockSpec((H, C), lambda i: (0, 0)),
        ],
        out_specs=[
            pl.BlockSpec((BN, H * C), lambda i: (i, 0)),
            pl.BlockSpec((BN, HP), lambda i: (i, 0)),
            pl.BlockSpec((BN, HP), lambda i: (i, 0)),
            pl.BlockSpec((1, HP), lambda i: (0, 0)),
            pl.BlockSpec((1, HP), lambda i: (0, 0)),
        ],
        out_shape=[
            jax.ShapeDtypeStruct((N, H * C), jnp.bfloat16),
            jax.ShapeDtypeStruct((N, HP), jnp.float32),
            jax.ShapeDtypeStruct((N, HP), jnp.float32),
            jax.ShapeDtypeStruct((1, HP), jnp.float32),
            jax.ShapeDtypeStruct((1, HP), jnp.float32),
        ],
    )(x, w, att_src, att_dst)


_MESH = plsc.VectorSubcoreMesh(core_axis_name="c", subcore_axis_name="s")


@functools.partial(
    pl.kernel,
    out_type=[
        jax.ShapeDtypeStruct((E, HP), jnp.float32),
        jax.ShapeDtypeStruct((NC, NPAD, HP), jnp.float32),
    ],
    mesh=_MESH,
    compiler_params=pltpu.CompilerParams(use_tc_tiling_on_sc=False),
    scratch_types=[
        pltpu.VMEM_SHARED((NPAD, HP), jnp.float32),
        pltpu.VMEM((CHA,), jnp.int32),
        pltpu.VMEM((CHA,), jnp.int32),
        pltpu.VMEM((CHA, HP), jnp.float32),
        pltpu.VMEM((CHA, HP), jnp.float32),
        pltpu.VMEM((CHA, HP), jnp.float32),
        pltpu.VMEM((1, HP), jnp.float32),
        pltpu.VMEM((1, HP), jnp.float32),
    ],
)
def _edge_pass_a(src3_hbm, dst3_hbm, ats_hbm, atd_hbm, ks_hbm, kd_hbm, z16_hbm,
                 e_hbm, den_hbm,
                 den_sh, sidx, didx, ag, bg, ebuf, ksb, kdb):
    c = lax.axis_index("c")
    s = lax.axis_index("s")
    wid = c * NS + s
    base = wid * EPW

    pltpu.sync_copy(z16_hbm.at[pl.ds(s * RPT, RPT), :],
                    den_sh.at[pl.ds(s * RPT, RPT), :])
    plsc.subcore_barrier()

    pltpu.sync_copy(ks_hbm, ksb)
    pltpu.sync_copy(kd_hbm, kdb)
    ksum = ksb[0] + kdb[0]
    kvec = jnp.maximum(ksum, NEG * ksum)

    def chunk(ci, carry):
        off = base + ci * CHA
        pltpu.sync_copy(src3_hbm.at[wid].at[ci], sidx)
        pltpu.sync_copy(dst3_hbm.at[wid].at[ci], didx)
        pltpu.sync_copy(ats_hbm.at[sidx], ag)
        pltpu.sync_copy(atd_hbm.at[didx], bg)

        def edge(i, _):
            a = ag[i] + bg[i]
            a = jnp.maximum(a, NEG * a)
            ebuf[i] = jnp.exp(a - kvec)
            return 0

        lax.fori_loop(0, CHA, edge, 0)
        pltpu.sync_copy(ebuf, e_hbm.at[pl.ds(off, CHA), :])
        pltpu.sync_copy(ebuf, den_sh.at[didx], add=True)
        return carry

    lax.fori_loop(0, NCHA, chunk, 0)

    plsc.subcore_barrier()
    pltpu.sync_copy(den_sh.at[pl.ds(s * RPT, RPT), :],
                    den_hbm.at[c].at[pl.ds(s * RPT, RPT), :])


def _rden_body(d_ref, r_ref):
    r_ref[...] = 1.0 / (d_ref[0] + d_ref[1] + 1e-16)


def _rden(den):
    return pl.pallas_call(
        _rden_body,
        out_shape=jax.ShapeDtypeStruct((NPAD, HP), jnp.float32),
    )(den)


@functools.partial(
    pl.kernel,
    out_type=jax.ShapeDtypeStruct((NC, NPAD, C), jnp.float32),
    mesh=_MESH,
    compiler_params=pltpu.CompilerParams(use_tc_tiling_on_sc=False,
                                         needs_layout_passes=False),
    scratch_types=[
        pltpu.VMEM_SHARED((NPAD, C), jnp.float32),
        pltpu.VMEM((CHB,), jnp.int32),
        pltpu.VMEM((CHB,), jnp.int32),
        pltpu.VMEM((CHB, HP), jnp.float32),
        pltpu.VMEM((CHB, HP), jnp.float32),
        pltpu.VMEM((CHB, H, C), jnp.bfloat16),
        pltpu.VMEM((CHB,), jnp.int32),
        pltpu.VMEM((CHB,), jnp.int32),
        pltpu.VMEM((CHB, HP), jnp.float32),
        pltpu.VMEM((CHB, HP), jnp.float32),
        pltpu.VMEM((CHB, H, C), jnp.bfloat16),
        pltpu.VMEM((CHB, C), jnp.float32),
        pltpu.SemaphoreType.DMA,
        pltpu.SemaphoreType.DMA,
        pltpu.SemaphoreType.DMA,
        pltpu.SemaphoreType.DMA,
        pltpu.SemaphoreType.DMA,
        pltpu.SemaphoreType.DMA,
    ],
)
def _edge_pass_b(src3_hbm, dst3_hbm, e_hbm, rd_hbm, xp_hbm, z128_hbm,
                 out_hbm,
                 out_sh, si0, di0, eb0, rb0, xb0, si1, di1, eb1, rb1, xb1,
                 msgb, es0, rs0, xs0, es1, rs1, xs1):
    c = lax.axis_index("c")
    s = lax.axis_index("s")
    wid = c * NS + s
    base = wid * EPW

    pltpu.sync_copy(z128_hbm.at[pl.ds(s * RPT, RPT), :],
                    out_sh.at[pl.ds(s * RPT, RPT), :])
    plsc.subcore_barrier()

    bufs = ((si0, di0, eb0, rb0, xb0, es0, rs0, xs0),
            (si1, di1, eb1, rb1, xb1, es1, rs1, xs1))

    def issue(ci, b):
        si, di, eb, rb, xb, es, rs, xs = b
        off = base + ci * CHB
        pltpu.sync_copy(src3_hbm.at[wid].at[ci], si)
        pltpu.sync_copy(dst3_hbm.at[wid].at[ci], di)
        pltpu.async_copy(e_hbm.at[pl.ds(off, CHB), :], eb, es)
        pltpu.async_copy(rd_hbm.at[di], rb, rs)
        pltpu.async_copy(xp_hbm.at[si], xb, xs)

    def drain(ci, b):
        si, di, eb, rb, xb, es, rs, xs = b
        off = base + ci * CHB
        pltpu.make_async_copy(e_hbm.at[pl.ds(off, CHB), :], eb, es).wait()
        pltpu.make_async_copy(rd_hbm.at[di], rb, rs).wait()
        pltpu.make_async_copy(xp_hbm.at[si], xb, xs).wait()

    def body(ci, b_cur, b_nxt):
        si, di, eb, rb, xb, es, rs, xs = b_cur

        @pl.when(ci + 1 < NCHB)
        def _():
            issue(ci + 1, b_nxt)

        drain(ci, b_cur)

        def edge(i, _):
            cfv = eb[i] * rb[i]
            accs = [jnp.zeros((16,), jnp.float32) for _ in range(8)]
            for h in range(H):
                bs = jnp.full((16,), cfv[h], jnp.float32)
                for cc in range(4):
                    v = xb[i, h, pl.ds(cc * 32, 32)]
                    vi = plsc.bitcast(v, jnp.int32)
                    lo = plsc.bitcast(jnp.left_shift(vi, 16), jnp.float32)
                    hi = plsc.bitcast(vi & jnp.int32(-65536), jnp.float32)
                    accs[2 * cc] = accs[2 * cc] + bs * lo
                    accs[2 * cc + 1] = accs[2 * cc + 1] + bs * hi
            for m in range(8):
                msgb[i, pl.ds(m * 16, 16)] = accs[m]
            return 0

        lax.fori_loop(0, CHB, edge, 0)
        pltpu.sync_copy(msgb, out_sh.at[di], add=True)

    issue(0, bufs[0])

    def pair(g, carry):
        body(2 * g, bufs[0], bufs[1])
        body(2 * g + 1, bufs[1], bufs[0])
        return carry

    lax.fori_loop(0, NCHB // 2, pair, 0)

    plsc.subcore_barrier()
    pltpu.sync_copy(out_sh.at[pl.ds(s * RPT, RPT), :],
                    out_hbm.at[c].at[pl.ds(s * RPT, RPT), :])


def _final_body(p_ref, b_ref, o_ref):
    o_ref[...] = (p_ref[0] + p_ref[1]) * (1.0 / H) + b_ref[...]


def _final(partials, bias2d):
    return pl.pallas_call(
        _final_body,
        grid=(N // BN,),
        in_specs=[
            pl.BlockSpec((NC, BN, C), lambda i: (0, i, 0)),
            pl.BlockSpec((1, C), lambda i: (0, 0)),
        ],
        out_specs=pl.BlockSpec((BN, C), lambda i: (i, 0)),
        out_shape=jax.ShapeDtypeStruct((N, C), jnp.float32),
    )(partials, bias2d)


def kernel(x, edge_index, W, att_src, att_dst, bias):
    src = edge_index[0]
    dst = edge_index[1]
    perm = jnp.asarray(_PERM)
    Wp = W.reshape(D, H, C)[:, :, perm].reshape(D, H * C)
    xpbf, ats, atd, ks, kd = _proj(x, Wp, att_src[:, perm], att_dst[:, perm])
    z16 = jnp.zeros((NPAD, HP), jnp.float32)
    z128 = jnp.zeros((NPAD, C), jnp.float32)
    srcA = src.reshape(NW, NCHA, CHA)
    dstA = dst.reshape(NW, NCHA, CHA)
    e, den = _edge_pass_a(srcA, dstA, ats, atd, ks, kd, z16)
    rden = _rden(den)
    srcB = src.reshape(NW, NCHB, CHB)
    dstB = dst.reshape(NW, NCHB, CHB)
    out_p = _edge_pass_b(srcB, dstB, e, rden, xpbf.reshape(N, H, C), z128)
    return _final(out_p, bias.reshape(1, C))


# speedup vs baseline: 42.7220x; 1.2816x over previous
import functools

import jax
import jax.numpy as jnp
import numpy as np
from jax import lax
from jax.experimental import pallas as pl
from jax.experimental.pallas import tpu as pltpu
from jax.experimental.pallas import tpu_sc as plsc

N = 10000
E = 320000
D = 128
H = 8
C = 128
HP = 16
NEG = 0.2

NC = 2
NS = 16
NW = NC * NS
EPW = E // NW
CHA = 80
NCHA = EPW // CHA
CHB = 40
NCHB = EPW // CHB
NPAD = 10240
RPT = NPAD // NS

BN = 2000

_PERM = np.arange(C).reshape(4, 2, 16).transpose(0, 2, 1).reshape(C)


def _proj_body(x_ref, w_ref, as_ref, ad_ref,
               xpbf_ref, ats_ref, atd_ref, ks_ref, kd_ref):
    i = pl.program_id(0)
    xp = jnp.dot(x_ref[...], w_ref[...], preferred_element_type=jnp.float32)
    xpbf_ref[...] = xp.astype(jnp.bfloat16)
    xph = xp.reshape(BN, H, C)
    asb = jnp.sum(xph * as_ref[...][None], axis=-1)
    adb = jnp.sum(xph * ad_ref[...][None], axis=-1)
    pad = jnp.zeros((BN, HP - H), jnp.float32)
    asbp = jnp.concatenate([asb, pad], axis=1)
    adbp = jnp.concatenate([adb, pad], axis=1)
    ats_ref[...] = asbp
    atd_ref[...] = adbp

    @pl.when(i == 0)
    def _():
        ks_ref[...] = jnp.full((1, HP), -1e30, jnp.float32)
        kd_ref[...] = jnp.full((1, HP), -1e30, jnp.float32)

    ks_ref[...] = jnp.maximum(ks_ref[...], jnp.max(asbp, axis=0, keepdims=True))
    kd_ref[...] = jnp.maximum(kd_ref[...], jnp.max(adbp, axis=0, keepdims=True))


def _proj(x, w, att_src, att_dst):
    return pl.pallas_call(
        _proj_body,
        grid=(N // BN,),
        in_specs=[
            pl.BlockSpec((BN, D), lambda i: (i, 0)),
            pl.BlockSpec((D, H * C), lambda i: (0, 0)),
            pl.BlockSpec((H, C), lambda i: (0, 0)),
            pl.BlockSpec((H, C), lambda i: (0, 0)),
        ],
        out_specs=[
            pl.BlockSpec((BN, H * C), lambda i: (i, 0)),
            pl.BlockSpec((BN, HP), lambda i: (i, 0)),
            pl.BlockSpec((BN, HP), lambda i: (i, 0)),
            pl.BlockSpec((1, HP), lambda i: (0, 0)),
            pl.BlockSpec((1, HP), lambda i: (0, 0)),
        ],
        out_shape=[
            jax.ShapeDtypeStruct((N, H * C), jnp.bfloat16),
            jax.ShapeDtypeStruct((N, HP), jnp.float32),
            jax.ShapeDtypeStruct((N, HP), jnp.float32),
            jax.ShapeDtypeStruct((1, HP), jnp.float32),
            jax.ShapeDtypeStruct((1, HP), jnp.float32),
        ],
    )(x, w, att_src, att_dst)


_MESH = plsc.VectorSubcoreMesh(core_axis_name="c", subcore_axis_name="s")


@functools.partial(
    pl.kernel,
    out_type=[
        jax.ShapeDtypeStruct((E, HP), jnp.float32),
        jax.ShapeDtypeStruct((NC, NPAD, HP), jnp.float32),
    ],
    mesh=_MESH,
    compiler_params=pltpu.CompilerParams(use_tc_tiling_on_sc=False),
    scratch_types=[
        pltpu.VMEM_SHARED((NPAD, HP), jnp.float32),
        pltpu.VMEM((CHA,), jnp.int32),
        pltpu.VMEM((CHA,), jnp.int32),
        pltpu.VMEM((CHA, HP), jnp.float32),
        pltpu.VMEM((CHA, HP), jnp.float32),
        pltpu.VMEM((CHA, HP), jnp.float32),
        pltpu.VMEM((CHA,), jnp.int32),
        pltpu.VMEM((CHA,), jnp.int32),
        pltpu.VMEM((CHA, HP), jnp.float32),
        pltpu.VMEM((CHA, HP), jnp.float32),
        pltpu.VMEM((CHA, HP), jnp.float32),
        pltpu.VMEM((1, HP), jnp.float32),
        pltpu.VMEM((1, HP), jnp.float32),
        pltpu.SemaphoreType.DMA,
        pltpu.SemaphoreType.DMA,
        pltpu.SemaphoreType.DMA,
        pltpu.SemaphoreType.DMA,
        pltpu.SemaphoreType.DMA,
        pltpu.SemaphoreType.DMA,
    ],
)
def _edge_pass_a(src3_hbm, dst3_hbm, ats_hbm, atd_hbm, ks_hbm, kd_hbm, z16_hbm,
                 e_hbm, den_hbm,
                 den_sh, si0, di0, ag0, bg0, eb0, si1, di1, ag1, bg1, eb1,
                 ksb, kdb, as0, bs0, ws0, as1, bs1, ws1):
    c = lax.axis_index("c")
    s = lax.axis_index("s")
    wid = c * NS + s
    base = wid * EPW

    pltpu.sync_copy(z16_hbm.at[pl.ds(s * RPT, RPT), :],
                    den_sh.at[pl.ds(s * RPT, RPT), :])
    plsc.subcore_barrier()

    pltpu.sync_copy(ks_hbm, ksb)
    pltpu.sync_copy(kd_hbm, kdb)
    ksum = ksb[0] + kdb[0]
    kvec = jnp.maximum(ksum, NEG * ksum)

    bufs = ((si0, di0, ag0, bg0, eb0, as0, bs0, ws0),
            (si1, di1, ag1, bg1, eb1, as1, bs1, ws1))

    def issue(ci, b):
        si, di, ag, bg, eb, asem, bsem, wsem = b
        pltpu.sync_copy(src3_hbm.at[wid].at[ci], si)
        pltpu.sync_copy(dst3_hbm.at[wid].at[ci], di)
        pltpu.async_copy(ats_hbm.at[si], ag, asem)
        pltpu.async_copy(atd_hbm.at[di], bg, bsem)

    def body(ci, g, b_cur, b_nxt):
        si, di, ag, bg, eb, asem, bsem, wsem = b_cur
        off = base + ci * CHA

        @pl.when(ci + 1 < NCHA)
        def _():
            issue(ci + 1, b_nxt)

        pltpu.make_async_copy(ats_hbm.at[si], ag, asem).wait()
        pltpu.make_async_copy(atd_hbm.at[di], bg, bsem).wait()

        def edge(i, _):
            a = ag[i] + bg[i]
            a = jnp.maximum(a, NEG * a)
            eb[i] = jnp.exp(a - kvec)
            return 0

        lax.fori_loop(0, CHA, edge, 0)
        pltpu.sync_copy(eb, e_hbm.at[pl.ds(off, CHA), :])
        pltpu.sync_copy(eb, den_sh.at[di], add=True)

    issue(0, bufs[0])
    body(0, 0, bufs[0], bufs[1])

    def pair(g, carry):
        body(2 * g + 1, g, bufs[1], bufs[0])
        body(2 * g + 2, g, bufs[0], bufs[1])
        return carry

    lax.fori_loop(0, NCHA // 2, pair, 0)

    plsc.subcore_barrier()
    pltpu.sync_copy(den_sh.at[pl.ds(s * RPT, RPT), :],
                    den_hbm.at[c].at[pl.ds(s * RPT, RPT), :])


def _rden_body(d_ref, r_ref):
    r_ref[...] = 1.0 / (d_ref[0] + d_ref[1] + 1e-16)


def _rden(den):
    return pl.pallas_call(
        _rden_body,
        out_shape=jax.ShapeDtypeStruct((NPAD, HP), jnp.float32),
    )(den)


@functools.partial(
    pl.kernel,
    out_type=jax.ShapeDtypeStruct((NC, NPAD, C), jnp.float32),
    mesh=_MESH,
    compiler_params=pltpu.CompilerParams(use_tc_tiling_on_sc=False,
                                         needs_layout_passes=False),
    scratch_types=[
        pltpu.VMEM_SHARED((NPAD, C), jnp.float32),
        pltpu.VMEM((2, CHB), jnp.int32),
        pltpu.VMEM((CHB, HP), jnp.float32),
        pltpu.VMEM((CHB, HP), jnp.float32),
        pltpu.VMEM((CHB, H, C), jnp.bfloat16),
        pltpu.VMEM((2, CHB), jnp.int32),
        pltpu.VMEM((CHB, HP), jnp.float32),
        pltpu.VMEM((CHB, HP), jnp.float32),
        pltpu.VMEM((CHB, H, C), jnp.bfloat16),
        pltpu.VMEM((CHB, C), jnp.float32),
        pltpu.SemaphoreType.DMA,
        pltpu.SemaphoreType.DMA,
        pltpu.SemaphoreType.DMA,
        pltpu.SemaphoreType.DMA,
        pltpu.SemaphoreType.DMA,
        pltpu.SemaphoreType.DMA,
    ],
)
def _edge_pass_b(sd4_hbm, e_hbm, rd_hbm, xp_hbm, z128_hbm,
                 out_hbm,
                 out_sh, sd0, eb0, rb0, xb0, sd1, eb1, rb1, xb1,
                 msgb, es0, rs0, xs0, es1, rs1, xs1):
    c = lax.axis_index("c")
    s = lax.axis_index("s")
    wid = c * NS + s
    base = wid * EPW

    pltpu.sync_copy(z128_hbm.at[pl.ds(s * RPT, RPT), :],
                    out_sh.at[pl.ds(s * RPT, RPT), :])
    plsc.subcore_barrier()

    bufs = ((sd0, eb0, rb0, xb0, es0, rs0, xs0),
            (sd1, eb1, rb1, xb1, es1, rs1, xs1))

    def issue(ci, b):
        sd, eb, rb, xb, es, rs, xs = b
        off = base + ci * CHB
        pltpu.sync_copy(sd4_hbm.at[wid].at[ci], sd)
        pltpu.async_copy(e_hbm.at[pl.ds(off, CHB), :], eb, es)
        pltpu.async_copy(rd_hbm.at[sd.at[1]], rb, rs)
        pltpu.async_copy(xp_hbm.at[sd.at[0]], xb, xs)

    def drain(ci, b):
        sd, eb, rb, xb, es, rs, xs = b
        off = base + ci * CHB
        pltpu.make_async_copy(e_hbm.at[pl.ds(off, CHB), :], eb, es).wait()
        pltpu.make_async_copy(rd_hbm.at[sd.at[1]], rb, rs).wait()
        pltpu.make_async_copy(xp_hbm.at[sd.at[0]], xb, xs).wait()

    def body(ci, b_cur, b_nxt):
        sd, eb, rb, xb, es, rs, xs = b_cur

        @pl.when(ci + 1 < NCHB)
        def _():
            issue(ci + 1, b_nxt)

        drain(ci, b_cur)

        def edge(i, _):
            cfv = eb[i] * rb[i]
            accs = [jnp.zeros((16,), jnp.float32) for _ in range(8)]
            for h in range(H):
                bs = jnp.full((16,), cfv[h], jnp.float32)
                for cc in range(4):
                    v = xb[i, h, pl.ds(cc * 32, 32)]
                    vi = plsc.bitcast(v, jnp.int32)
                    lo = plsc.bitcast(jnp.left_shift(vi, 16), jnp.float32)
                    hi = plsc.bitcast(vi & jnp.int32(-65536), jnp.float32)
                    accs[2 * cc] = accs[2 * cc] + bs * lo
                    accs[2 * cc + 1] = accs[2 * cc + 1] + bs * hi
            for m in range(8):
                msgb[i, pl.ds(m * 16, 16)] = accs[m]
            return 0

        lax.fori_loop(0, CHB, edge, 0, unroll=2)
        pltpu.sync_copy(msgb, out_sh.at[sd.at[1]], add=True)

    issue(0, bufs[0])

    def pair(g, carry):
        body(2 * g, bufs[0], bufs[1])
        body(2 * g + 1, bufs[1], bufs[0])
        return carry

    lax.fori_loop(0, NCHB // 2, pair, 0)

    plsc.subcore_barrier()
    pltpu.sync_copy(out_sh.at[pl.ds(s * RPT, RPT), :],
                    out_hbm.at[c].at[pl.ds(s * RPT, RPT), :])


def _final_body(p_ref, b_ref, o_ref):
    o_ref[...] = (p_ref[0] + p_ref[1]) * (1.0 / H) + b_ref[...]


def _final(partials, bias2d):
    return pl.pallas_call(
        _final_body,
        grid=(N // BN,),
        in_specs=[
            pl.BlockSpec((NC, BN, C), lambda i: (0, i, 0)),
            pl.BlockSpec((1, C), lambda i: (0, 0)),
        ],
        out_specs=pl.BlockSpec((BN, C), lambda i: (i, 0)),
        out_shape=jax.ShapeDtypeStruct((N, C), jnp.float32),
    )(partials, bias2d)


def kernel(x, edge_index, W, att_src, att_dst, bias):
    src = edge_index[0]
    dst = edge_index[1]
    perm = jnp.asarray(_PERM)
    Wp = W.reshape(D, H, C)[:, :, perm].reshape(D, H * C)
    xpbf, ats, atd, ks, kd = _proj(x, Wp, att_src[:, perm], att_dst[:, perm])
    z16 = jnp.zeros((NPAD, HP), jnp.float32)
    z128 = jnp.zeros((NPAD, C), jnp.float32)
    srcA = src.reshape(NW, NCHA, CHA)
    dstA = dst.reshape(NW, NCHA, CHA)
    e, den = _edge_pass_a(srcA, dstA, ats, atd, ks, kd, z16)
    rden = _rden(den)
    sd4 = jnp.stack([src.reshape(NW, NCHB, CHB),
                     dst.reshape(NW, NCHB, CHB)], axis=2)
    out_p = _edge_pass_b(sd4, e, rden, xpbf.reshape(N, H, C), z128)
    return _final(out_p, bias.reshape(1, C))


# speedup vs baseline: 43.6995x; 1.0229x over previous
import functools

import jax
import jax.numpy as jnp
import numpy as np
from jax import lax
from jax.experimental import pallas as pl
from jax.experimental.pallas import tpu as pltpu
from jax.experimental.pallas import tpu_sc as plsc

N = 10000
E = 320000
D = 128
H = 8
C = 128
HP = 16
NEG = 0.2

NC = 2
NS = 16
NW = NC * NS
EPW = E // NW
CHA = 80
NCHA = EPW // CHA
CHB = 40
NCHB = EPW // CHB
NPAD = 10240
RPT = NPAD // NS

BN = 2000

_PERM = np.arange(C).reshape(4, 2, 16).transpose(0, 2, 1).reshape(C)


def _proj_body(x_ref, w_ref, as_ref, ad_ref,
               xpbf_ref, ats_ref, atd_ref, ks_ref, kd_ref):
    i = pl.program_id(0)
    xp = jnp.dot(x_ref[...], w_ref[...], preferred_element_type=jnp.float32)
    xpbf_ref[...] = xp.astype(jnp.bfloat16)
    xph = xp.reshape(BN, H, C)
    asb = jnp.sum(xph * as_ref[...][None], axis=-1)
    adb = jnp.sum(xph * ad_ref[...][None], axis=-1)
    pad = jnp.zeros((BN, HP - H), jnp.float32)
    asbp = jnp.concatenate([asb, pad], axis=1)
    adbp = jnp.concatenate([adb, pad], axis=1)
    ats_ref[...] = asbp
    atd_ref[...] = adbp

    @pl.when(i == 0)
    def _():
        ks_ref[...] = jnp.full((1, HP), -1e30, jnp.float32)
        kd_ref[...] = jnp.full((1, HP), -1e30, jnp.float32)

    ks_ref[...] = jnp.maximum(ks_ref[...], jnp.max(asbp, axis=0, keepdims=True))
    kd_ref[...] = jnp.maximum(kd_ref[...], jnp.max(adbp, axis=0, keepdims=True))


def _proj(x, w, att_src, att_dst):
    return pl.pallas_call(
        _proj_body,
        grid=(N // BN,),
        in_specs=[
            pl.BlockSpec((BN, D), lambda i: (i, 0)),
            pl.BlockSpec((D, H * C), lambda i: (0, 0)),
            pl.BlockSpec((H, C), lambda i: (0, 0)),
            pl.BlockSpec((H, C), lambda i: (0, 0)),
        ],
        out_specs=[
            pl.BlockSpec((BN, H * C), lambda i: (i, 0)),
            pl.BlockSpec((BN, HP), lambda i: (i, 0)),
            pl.BlockSpec((BN, HP), lambda i: (i, 0)),
            pl.BlockSpec((1, HP), lambda i: (0, 0)),
            pl.BlockSpec((1, HP), lambda i: (0, 0)),
        ],
        out_shape=[
            jax.ShapeDtypeStruct((N, H * C), jnp.bfloat16),
            jax.ShapeDtypeStruct((N, HP), jnp.float32),
            jax.ShapeDtypeStruct((N, HP), jnp.float32),
            jax.ShapeDtypeStruct((1, HP), jnp.float32),
            jax.ShapeDtypeStruct((1, HP), jnp.float32),
        ],
    )(x, w, att_src, att_dst)


_MESH = plsc.VectorSubcoreMesh(core_axis_name="c", subcore_axis_name="s")


@functools.partial(
    pl.kernel,
    out_type=[
        jax.ShapeDtypeStruct((E, HP), jnp.float32),
        jax.ShapeDtypeStruct((NC, NPAD, HP), jnp.float32),
    ],
    mesh=_MESH,
    compiler_params=pltpu.CompilerParams(use_tc_tiling_on_sc=False),
    scratch_types=[
        pltpu.VMEM_SHARED((NPAD, HP), jnp.float32),
        pltpu.VMEM((2, CHA), jnp.int32),
        pltpu.VMEM((CHA, HP), jnp.float32),
        pltpu.VMEM((CHA, HP), jnp.float32),
        pltpu.VMEM((CHA, HP), jnp.float32),
        pltpu.VMEM((2, CHA), jnp.int32),
        pltpu.VMEM((CHA, HP), jnp.float32),
        pltpu.VMEM((CHA, HP), jnp.float32),
        pltpu.VMEM((CHA, HP), jnp.float32),
        pltpu.VMEM((1, HP), jnp.float32),
        pltpu.VMEM((1, HP), jnp.float32),
        pltpu.SemaphoreType.DMA,
        pltpu.SemaphoreType.DMA,
        pltpu.SemaphoreType.DMA,
        pltpu.SemaphoreType.DMA,
    ],
)
def _edge_pass_a(sdA_hbm, ats_hbm, atd_hbm, ks_hbm, kd_hbm, z16_hbm,
                 e_hbm, den_hbm,
                 den_sh, sd0, ag0, bg0, eb0, sd1, ag1, bg1, eb1,
                 ksb, kdb, as0, bs0, as1, bs1):
    c = lax.axis_index("c")
    s = lax.axis_index("s")
    wid = c * NS + s
    base = wid * EPW

    pltpu.sync_copy(z16_hbm.at[pl.ds(s * RPT, RPT), :],
                    den_sh.at[pl.ds(s * RPT, RPT), :])
    plsc.subcore_barrier()

    pltpu.sync_copy(ks_hbm, ksb)
    pltpu.sync_copy(kd_hbm, kdb)
    ksum = ksb[0] + kdb[0]
    kvec = jnp.maximum(ksum, NEG * ksum)

    bufs = ((sd0, ag0, bg0, eb0, as0, bs0),
            (sd1, ag1, bg1, eb1, as1, bs1))

    def issue(ci, b):
        sd, ag, bg, eb, asem, bsem = b
        pltpu.sync_copy(sdA_hbm.at[wid].at[ci], sd)
        pltpu.async_copy(ats_hbm.at[sd.at[0]], ag, asem)
        pltpu.async_copy(atd_hbm.at[sd.at[1]], bg, bsem)

    def body(ci, b_cur, b_nxt):
        sd, ag, bg, eb, asem, bsem = b_cur
        off = base + ci * CHA

        @pl.when(ci + 1 < NCHA)
        def _():
            issue(ci + 1, b_nxt)

        pltpu.make_async_copy(ats_hbm.at[sd.at[0]], ag, asem).wait()
        pltpu.make_async_copy(atd_hbm.at[sd.at[1]], bg, bsem).wait()

        def edge(i, _):
            a = ag[i] + bg[i]
            a = jnp.maximum(a, NEG * a)
            eb[i] = jnp.exp(a - kvec)
            return 0

        lax.fori_loop(0, CHA, edge, 0)
        pltpu.sync_copy(eb, e_hbm.at[pl.ds(off, CHA), :])
        pltpu.sync_copy(eb, den_sh.at[sd.at[1]], add=True)

    issue(0, bufs[0])
    body(0, bufs[0], bufs[1])

    def pair(g, carry):
        body(2 * g + 1, bufs[1], bufs[0])
        body(2 * g + 2, bufs[0], bufs[1])
        return carry

    lax.fori_loop(0, NCHA // 2, pair, 0)

    plsc.subcore_barrier()
    pltpu.sync_copy(den_sh.at[pl.ds(s * RPT, RPT), :],
                    den_hbm.at[c].at[pl.ds(s * RPT, RPT), :])


def _rden_body(d_ref, r_ref):
    r_ref[...] = 1.0 / (d_ref[0] + d_ref[1] + 1e-16)


def _rden(den):
    return pl.pallas_call(
        _rden_body,
        out_shape=jax.ShapeDtypeStruct((NPAD, HP), jnp.float32),
    )(den)


@functools.partial(
    pl.kernel,
    out_type=jax.ShapeDtypeStruct((NC, NPAD, C), jnp.float32),
    mesh=_MESH,
    compiler_params=pltpu.CompilerParams(use_tc_tiling_on_sc=False,
                                         needs_layout_passes=False),
    scratch_types=[
        pltpu.VMEM_SHARED((NPAD, C), jnp.float32),
        pltpu.VMEM((2, CHB), jnp.int32),
        pltpu.VMEM((CHB, HP), jnp.float32),
        pltpu.VMEM((CHB, HP), jnp.float32),
        pltpu.VMEM((CHB, H, C), jnp.bfloat16),
        pltpu.VMEM((2, CHB), jnp.int32),
        pltpu.VMEM((CHB, HP), jnp.float32),
        pltpu.VMEM((CHB, HP), jnp.float32),
        pltpu.VMEM((CHB, H, C), jnp.bfloat16),
        pltpu.VMEM((CHB, C), jnp.float32),
        pltpu.SemaphoreType.DMA,
        pltpu.SemaphoreType.DMA,
        pltpu.SemaphoreType.DMA,
        pltpu.SemaphoreType.DMA,
        pltpu.SemaphoreType.DMA,
        pltpu.SemaphoreType.DMA,
    ],
)
def _edge_pass_b(sd4_hbm, e_hbm, rd_hbm, xp_hbm, z128_hbm,
                 out_hbm,
                 out_sh, sd0, eb0, rb0, xb0, sd1, eb1, rb1, xb1,
                 msgb, es0, rs0, xs0, es1, rs1, xs1):
    c = lax.axis_index("c")
    s = lax.axis_index("s")
    wid = c * NS + s
    base = wid * EPW

    pltpu.sync_copy(z128_hbm.at[pl.ds(s * RPT, RPT), :],
                    out_sh.at[pl.ds(s * RPT, RPT), :])
    plsc.subcore_barrier()

    bufs = ((sd0, eb0, rb0, xb0, es0, rs0, xs0),
            (sd1, eb1, rb1, xb1, es1, rs1, xs1))

    def issue(ci, b):
        sd, eb, rb, xb, es, rs, xs = b
        off = base + ci * CHB
        pltpu.sync_copy(sd4_hbm.at[wid].at[ci], sd)
        pltpu.async_copy(e_hbm.at[pl.ds(off, CHB), :], eb, es)
        pltpu.async_copy(rd_hbm.at[sd.at[1]], rb, rs)
        pltpu.async_copy(xp_hbm.at[sd.at[0]], xb, xs)

    def drain(ci, b):
        sd, eb, rb, xb, es, rs, xs = b
        off = base + ci * CHB
        pltpu.make_async_copy(e_hbm.at[pl.ds(off, CHB), :], eb, es).wait()
        pltpu.make_async_copy(rd_hbm.at[sd.at[1]], rb, rs).wait()
        pltpu.make_async_copy(xp_hbm.at[sd.at[0]], xb, xs).wait()

    def body(ci, b_cur, b_nxt):
        sd, eb, rb, xb, es, rs, xs = b_cur

        @pl.when(ci + 1 < NCHB)
        def _():
            issue(ci + 1, b_nxt)

        drain(ci, b_cur)

        def edge(i, _):
            cfv = eb[i] * rb[i]
            accs = [jnp.zeros((16,), jnp.float32) for _ in range(8)]
            for h in range(H):
                bs = jnp.full((16,), cfv[h], jnp.float32)
                for cc in range(4):
                    v = xb[i, h, pl.ds(cc * 32, 32)]
                    vi = plsc.bitcast(v, jnp.int32)
                    lo = plsc.bitcast(jnp.left_shift(vi, 16), jnp.float32)
                    hi = plsc.bitcast(vi & jnp.int32(-65536), jnp.float32)
                    accs[2 * cc] = accs[2 * cc] + bs * lo
                    accs[2 * cc + 1] = accs[2 * cc + 1] + bs * hi
            for m in range(8):
                msgb[i, pl.ds(m * 16, 16)] = accs[m]
            return 0

        lax.fori_loop(0, CHB, edge, 0, unroll=2)
        pltpu.sync_copy(msgb, out_sh.at[sd.at[1]], add=True)

    issue(0, bufs[0])

    def pair(g, carry):
        body(2 * g, bufs[0], bufs[1])
        body(2 * g + 1, bufs[1], bufs[0])
        return carry

    lax.fori_loop(0, NCHB // 2, pair, 0)

    plsc.subcore_barrier()
    pltpu.sync_copy(out_sh.at[pl.ds(s * RPT, RPT), :],
                    out_hbm.at[c].at[pl.ds(s * RPT, RPT), :])


def _final_body(p_ref, b_ref, o_ref):
    o_ref[...] = (p_ref[0] + p_ref[1]) * (1.0 / H) + b_ref[...]


def _final(partials, bias2d):
    return pl.pallas_call(
        _final_body,
        grid=(N // BN,),
        in_specs=[
            pl.BlockSpec((NC, BN, C), lambda i: (0, i, 0)),
            pl.BlockSpec((1, C), lambda i: (0, 0)),
        ],
        out_specs=pl.BlockSpec((BN, C), lambda i: (i, 0)),
        out_shape=jax.ShapeDtypeStruct((N, C), jnp.float32),
    )(partials, bias2d)


def kernel(x, edge_index, W, att_src, att_dst, bias):
    src = edge_index[0]
    dst = edge_index[1]
    perm = jnp.asarray(_PERM)
    Wp = W.reshape(D, H, C)[:, :, perm].reshape(D, H * C)
    xpbf, ats, atd, ks, kd = _proj(x, Wp, att_src[:, perm], att_dst[:, perm])
    z16 = jnp.zeros((NPAD, HP), jnp.float32)
    z128 = jnp.zeros((NPAD, C), jnp.float32)
    sdA = jnp.stack([src.reshape(NW, NCHA, CHA),
                     dst.reshape(NW, NCHA, CHA)], axis=2)
    e, den = _edge_pass_a(sdA, ats, atd, ks, kd, z16)
    rden = _rden(den)
    sd4 = jnp.stack([src.reshape(NW, NCHB, CHB),
                     dst.reshape(NW, NCHB, CHB)], axis=2)
    out_p = _edge_pass_b(sd4, e, rden, xpbf.reshape(N, H, C), z128)
    return _final(out_p, bias.reshape(1, C))


# speedup vs baseline: 47.1021x; 1.0779x over previous
import functools

import jax
import jax.numpy as jnp
import numpy as np
from jax import lax
from jax.experimental import pallas as pl
from jax.experimental.pallas import tpu as pltpu
from jax.experimental.pallas import tpu_sc as plsc

N = 10000
E = 320000
D = 128
H = 8
C = 128
HP = 16
NEG = 0.2

NC = 2
NS = 16
NW = NC * NS
EPW = E // NW
CHA = 80
NCHA = EPW // CHA
CHB = 40
NCHB = EPW // CHB
NPAD = 10240
RPT = NPAD // NS

BN = 2000

_PERM = np.arange(C).reshape(4, 2, 16).transpose(0, 2, 1).reshape(C)


def _proj_body(x_ref, w_ref, as_ref, ad_ref,
               xpbf_ref, ats_ref, atd_ref, ks_ref, kd_ref):
    i = pl.program_id(0)
    xp = jnp.dot(x_ref[...], w_ref[...], preferred_element_type=jnp.float32)
    xpbf_ref[...] = xp.astype(jnp.bfloat16)
    xph = xp.reshape(BN, H, C)
    asb = jnp.sum(xph * as_ref[...][None], axis=-1)
    adb = jnp.sum(xph * ad_ref[...][None], axis=-1)
    pad = jnp.zeros((BN, HP - H), jnp.float32)
    asbp = jnp.concatenate([asb, pad], axis=1)
    adbp = jnp.concatenate([adb, pad], axis=1)
    ats_ref[...] = asbp
    atd_ref[...] = adbp

    @pl.when(i == 0)
    def _():
        ks_ref[...] = jnp.full((1, HP), -1e30, jnp.float32)
        kd_ref[...] = jnp.full((1, HP), -1e30, jnp.float32)

    ks_ref[...] = jnp.maximum(ks_ref[...], jnp.max(asbp, axis=0, keepdims=True))
    kd_ref[...] = jnp.maximum(kd_ref[...], jnp.max(adbp, axis=0, keepdims=True))


def _proj(x, w, att_src, att_dst):
    return pl.pallas_call(
        _proj_body,
        grid=(N // BN,),
        in_specs=[
            pl.BlockSpec((BN, D), lambda i: (i, 0)),
            pl.BlockSpec((D, H * C), lambda i: (0, 0)),
            pl.BlockSpec((H, C), lambda i: (0, 0)),
            pl.BlockSpec((H, C), lambda i: (0, 0)),
        ],
        out_specs=[
            pl.BlockSpec((BN, H * C), lambda i: (i, 0)),
            pl.BlockSpec((BN, HP), lambda i: (i, 0)),
            pl.BlockSpec((BN, HP), lambda i: (i, 0)),
            pl.BlockSpec((1, HP), lambda i: (0, 0)),
            pl.BlockSpec((1, HP), lambda i: (0, 0)),
        ],
        out_shape=[
            jax.ShapeDtypeStruct((N, H * C), jnp.bfloat16),
            jax.ShapeDtypeStruct((N, HP), jnp.float32),
            jax.ShapeDtypeStruct((N, HP), jnp.float32),
            jax.ShapeDtypeStruct((1, HP), jnp.float32),
            jax.ShapeDtypeStruct((1, HP), jnp.float32),
        ],
    )(x, w, att_src, att_dst)


_MESH = plsc.VectorSubcoreMesh(core_axis_name="c", subcore_axis_name="s")


@functools.partial(
    pl.kernel,
    out_type=[
        jax.ShapeDtypeStruct((E, HP), jnp.float32),
        jax.ShapeDtypeStruct((NC, NPAD, HP), jnp.float32),
    ],
    mesh=_MESH,
    compiler_params=pltpu.CompilerParams(use_tc_tiling_on_sc=False),
    scratch_types=[
        pltpu.VMEM_SHARED((NPAD, HP), jnp.float32),
        pltpu.VMEM((2, CHA), jnp.int32),
        pltpu.VMEM((CHA, HP), jnp.float32),
        pltpu.VMEM((CHA, HP), jnp.float32),
        pltpu.VMEM((CHA, HP), jnp.float32),
        pltpu.VMEM((2, CHA), jnp.int32),
        pltpu.VMEM((CHA, HP), jnp.float32),
        pltpu.VMEM((CHA, HP), jnp.float32),
        pltpu.VMEM((CHA, HP), jnp.float32),
        pltpu.VMEM((1, HP), jnp.float32),
        pltpu.VMEM((1, HP), jnp.float32),
        pltpu.SemaphoreType.DMA,
        pltpu.SemaphoreType.DMA,
        pltpu.SemaphoreType.DMA,
        pltpu.SemaphoreType.DMA,
    ],
)
def _edge_pass_a(sdA_hbm, ats_hbm, atd_hbm, ks_hbm, kd_hbm, z16_hbm,
                 e_hbm, den_hbm,
                 den_sh, sd0, ag0, bg0, eb0, sd1, ag1, bg1, eb1,
                 ksb, kdb, as0, bs0, as1, bs1):
    c = lax.axis_index("c")
    s = lax.axis_index("s")
    wid = c * NS + s
    base = wid * EPW

    pltpu.sync_copy(z16_hbm.at[pl.ds(s * RPT, RPT), :],
                    den_sh.at[pl.ds(s * RPT, RPT), :])
    plsc.subcore_barrier()

    pltpu.sync_copy(ks_hbm, ksb)
    pltpu.sync_copy(kd_hbm, kdb)
    ksum = ksb[0] + kdb[0]
    kvec = jnp.maximum(ksum, NEG * ksum)

    bufs = ((sd0, ag0, bg0, eb0, as0, bs0),
            (sd1, ag1, bg1, eb1, as1, bs1))

    def issue(ci, b):
        sd, ag, bg, eb, asem, bsem = b
        pltpu.sync_copy(sdA_hbm.at[wid].at[ci], sd)
        pltpu.async_copy(ats_hbm.at[sd.at[0]], ag, asem)
        pltpu.async_copy(atd_hbm.at[sd.at[1]], bg, bsem)

    def body(ci, b_cur, b_nxt):
        sd, ag, bg, eb, asem, bsem = b_cur
        off = base + ci * CHA

        @pl.when(ci + 1 < NCHA)
        def _():
            issue(ci + 1, b_nxt)

        pltpu.make_async_copy(ats_hbm.at[sd.at[0]], ag, asem).wait()
        pltpu.make_async_copy(atd_hbm.at[sd.at[1]], bg, bsem).wait()

        def edge(i, _):
            a = ag[i] + bg[i]
            a = jnp.maximum(a, NEG * a)
            eb[i] = jnp.exp(a - kvec)
            return 0

        lax.fori_loop(0, CHA, edge, 0)
        pltpu.sync_copy(eb, e_hbm.at[pl.ds(off, CHA), :])
        pltpu.sync_copy(eb, den_sh.at[sd.at[1]], add=True)

    issue(0, bufs[0])
    body(0, bufs[0], bufs[1])

    def pair(g, carry):
        body(2 * g + 1, bufs[1], bufs[0])
        body(2 * g + 2, bufs[0], bufs[1])
        return carry

    lax.fori_loop(0, NCHA // 2, pair, 0)

    plsc.subcore_barrier()
    pltpu.sync_copy(den_sh.at[pl.ds(s * RPT, RPT), :],
                    den_hbm.at[c].at[pl.ds(s * RPT, RPT), :])


def _rden_body(d_ref, r_ref):
    r_ref[...] = 1.0 / (d_ref[0] + d_ref[1] + 1e-16)


def _rden(den):
    return pl.pallas_call(
        _rden_body,
        out_shape=jax.ShapeDtypeStruct((NPAD, HP), jnp.float32),
    )(den)


@functools.partial(
    pl.kernel,
    out_type=jax.ShapeDtypeStruct((NC, NPAD, C), jnp.float32),
    mesh=_MESH,
    compiler_params=pltpu.CompilerParams(use_tc_tiling_on_sc=False,
                                         needs_layout_passes=False),
    scratch_types=[
        pltpu.VMEM_SHARED((NPAD, C), jnp.float32),
        pltpu.VMEM((2, CHB), jnp.int32),
        pltpu.VMEM((CHB, HP), jnp.float32),
        pltpu.VMEM((CHB, HP), jnp.float32),
        pltpu.VMEM((CHB, H, C), jnp.bfloat16),
        pltpu.VMEM((2, CHB), jnp.int32),
        pltpu.VMEM((CHB, HP), jnp.float32),
        pltpu.VMEM((CHB, HP), jnp.float32),
        pltpu.VMEM((CHB, H, C), jnp.bfloat16),
        pltpu.VMEM((CHB, C), jnp.float32),
        pltpu.SemaphoreType.DMA,
        pltpu.SemaphoreType.DMA,
        pltpu.SemaphoreType.DMA,
        pltpu.SemaphoreType.DMA,
        pltpu.SemaphoreType.DMA,
        pltpu.SemaphoreType.DMA,
    ],
)
def _edge_pass_b(sd4_hbm, e_hbm, rd_hbm, xp_hbm, z128_hbm,
                 out_hbm,
                 out_sh, sd0, eb0, rb0, xb0, sd1, eb1, rb1, xb1,
                 msgb, es0, rs0, xs0, es1, rs1, xs1):
    c = lax.axis_index("c")
    s = lax.axis_index("s")
    wid = c * NS + s
    base = wid * EPW

    pltpu.sync_copy(z128_hbm.at[pl.ds(s * RPT, RPT), :],
                    out_sh.at[pl.ds(s * RPT, RPT), :])
    plsc.subcore_barrier()

    bufs = ((sd0, eb0, rb0, xb0, es0, rs0, xs0),
            (sd1, eb1, rb1, xb1, es1, rs1, xs1))

    def issue(ci, b):
        sd, eb, rb, xb, es, rs, xs = b
        off = base + ci * CHB
        pltpu.sync_copy(sd4_hbm.at[wid].at[ci], sd)
        pltpu.async_copy(e_hbm.at[pl.ds(off, CHB), :], eb, es)
        pltpu.async_copy(rd_hbm.at[sd.at[1]], rb, rs)
        pltpu.async_copy(xp_hbm.at[sd.at[0]], xb, xs)

    def drain(ci, b):
        sd, eb, rb, xb, es, rs, xs = b
        off = base + ci * CHB
        pltpu.make_async_copy(e_hbm.at[pl.ds(off, CHB), :], eb, es).wait()
        pltpu.make_async_copy(rd_hbm.at[sd.at[1]], rb, rs).wait()
        pltpu.make_async_copy(xp_hbm.at[sd.at[0]], xb, xs).wait()

    def body(ci, b_cur, b_nxt):
        sd, eb, rb, xb, es, rs, xs = b_cur

        @pl.when(ci + 1 < NCHB)
        def _():
            issue(ci + 1, b_nxt)

        drain(ci, b_cur)

        def edge(i, _):
            cfv = eb[i] * rb[i]
            accs = [jnp.zeros((16,), jnp.float32) for _ in range(8)]
            for h in range(H):
                bs = jnp.full((16,), cfv[h], jnp.float32)
                for cc in range(4):
                    v = xb[i, h, pl.ds(cc * 32, 32)]
                    vi = plsc.bitcast(v, jnp.int32)
                    lo = plsc.bitcast(jnp.left_shift(vi, 16), jnp.float32)
                    hi = plsc.bitcast(vi, jnp.float32)
                    accs[2 * cc] = accs[2 * cc] + bs * lo
                    accs[2 * cc + 1] = accs[2 * cc + 1] + bs * hi
            for m in range(8):
                msgb[i, pl.ds(m * 16, 16)] = accs[m]
            return 0

        lax.fori_loop(0, CHB, edge, 0, unroll=2)
        pltpu.sync_copy(msgb, out_sh.at[sd.at[1]], add=True)

    issue(0, bufs[0])

    def pair(g, carry):
        body(2 * g, bufs[0], bufs[1])
        body(2 * g + 1, bufs[1], bufs[0])
        return carry

    lax.fori_loop(0, NCHB // 2, pair, 0)

    plsc.subcore_barrier()
    pltpu.sync_copy(out_sh.at[pl.ds(s * RPT, RPT), :],
                    out_hbm.at[c].at[pl.ds(s * RPT, RPT), :])


def _final_body(p_ref, b_ref, o_ref):
    o_ref[...] = (p_ref[0] + p_ref[1]) * (1.0 / H) + b_ref[...]


def _final(partials, bias2d):
    return pl.pallas_call(
        _final_body,
        grid=(N // BN,),
        in_specs=[
            pl.BlockSpec((NC, BN, C), lambda i: (0, i, 0)),
            pl.BlockSpec((1, C), lambda i: (0, 0)),
        ],
        out_specs=pl.BlockSpec((BN, C), lambda i: (i, 0)),
        out_shape=jax.ShapeDtypeStruct((N, C), jnp.float32),
    )(partials, bias2d)


def kernel(x, edge_index, W, att_src, att_dst, bias):
    src = edge_index[0]
    dst = edge_index[1]
    perm = jnp.asarray(_PERM)
    Wp = W.reshape(D, H, C)[:, :, perm].reshape(D, H * C)
    xpbf, ats, atd, ks, kd = _proj(x, Wp, att_src[:, perm], att_dst[:, perm])
    z16 = jnp.zeros((NPAD, HP), jnp.float32)
    z128 = jnp.zeros((NPAD, C), jnp.float32)
    sdA = jnp.stack([src.reshape(NW, NCHA, CHA),
                     dst.reshape(NW, NCHA, CHA)], axis=2)
    e, den = _edge_pass_a(sdA, ats, atd, ks, kd, z16)
    rden = _rden(den)
    sd4 = jnp.stack([src.reshape(NW, NCHB, CHB),
                     dst.reshape(NW, NCHB, CHB)], axis=2)
    out_p = _edge_pass_b(sd4, e, rden, xpbf.reshape(N, H, C), z128)
    return _final(out_p, bias.reshape(1, C))
